# baseline (speedup 1.0000x reference)
import jax
import jax.numpy as jnp
from jax.experimental import pallas as pl
from jax.experimental.pallas import tpu as pltpu

DIM = 192
TIME = 32
HW = 1024
D6 = DIM // 6
DSH = 2 * D6
S_TOT = 16
NSLOT = 8


def _rope_body(tidx_ref, ttab_c_ref, ttab_s_ref, spat_c_ref, spat_s_ref,
               cos_hbm, sin_hbm, cbuf, sbuf, csem, ssem):
    i = pl.program_id(0)
    n = pl.num_programs(0)
    slot = jax.lax.rem(i, NSLOT)

    def item_dst(hbm, item):
        b = jax.lax.div(item, S_TOT)
        s = jax.lax.rem(item, S_TOT)
        return hbm.at[b, pl.ds(s * HW, HW), :]

    @pl.when(i >= NSLOT)
    def _():
        prev = i - NSLOT
        pltpu.make_async_copy(cbuf.at[slot], item_dst(cos_hbm, prev),
                              csem.at[slot]).wait()
        pltpu.make_async_copy(sbuf.at[slot], item_dst(sin_hbm, prev),
                              ssem.at[slot]).wait()

    idx = tidx_ref[jax.lax.div(i, S_TOT), jax.lax.rem(i, S_TOT)]
    cbuf[slot] = spat_c_ref[...] + ttab_c_ref[pl.ds(idx, 1), :]
    sbuf[slot] = spat_s_ref[...] + ttab_s_ref[pl.ds(idx, 1), :]
    pltpu.make_async_copy(cbuf.at[slot], item_dst(cos_hbm, i),
                          csem.at[slot]).start()
    pltpu.make_async_copy(sbuf.at[slot], item_dst(sin_hbm, i),
                          ssem.at[slot]).start()

    @pl.when(i == n - 1)
    def _():
        for k in range(NSLOT):
            item = n - NSLOT + k
            sl = jax.lax.rem(item, NSLOT)
            pltpu.make_async_copy(cbuf.at[sl], item_dst(cos_hbm, item),
                                  csem.at[sl]).wait()
            pltpu.make_async_copy(sbuf.at[sl], item_dst(sin_hbm, item),
                                  ssem.at[sl]).wait()


def kernel(t_idxs, cos_t, sin_t, cos_s, sin_s):
    B, S = t_idxs.shape
    zt = jnp.zeros((TIME, DSH), jnp.float32)
    ttab_c = jnp.concatenate([cos_t, zt, cos_t, zt], axis=1)
    ttab_s = jnp.concatenate([sin_t, zt, sin_t, zt], axis=1)
    zs = jnp.zeros((HW, D6), jnp.float32)
    spat_c = jnp.concatenate([zs, cos_s, zs, cos_s], axis=1)
    spat_s = jnp.concatenate([zs, sin_s, zs, sin_s], axis=1)

    grid_spec = pltpu.PrefetchScalarGridSpec(
        num_scalar_prefetch=1,
        grid=(B * S,),
        in_specs=[
            pl.BlockSpec((TIME, DIM), lambda i, tidx: (0, 0)),
            pl.BlockSpec((TIME, DIM), lambda i, tidx: (0, 0)),
            pl.BlockSpec((HW, DIM), lambda i, tidx: (0, 0)),
            pl.BlockSpec((HW, DIM), lambda i, tidx: (0, 0)),
        ],
        out_specs=[
            pl.BlockSpec(memory_space=pltpu.HBM),
            pl.BlockSpec(memory_space=pltpu.HBM),
        ],
        scratch_shapes=[
            pltpu.VMEM((NSLOT, HW, DIM), jnp.float32),
            pltpu.VMEM((NSLOT, HW, DIM), jnp.float32),
            pltpu.SemaphoreType.DMA((NSLOT,)),
            pltpu.SemaphoreType.DMA((NSLOT,)),
        ],
    )
    out_shape = jax.ShapeDtypeStruct((B, S * HW, DIM), jnp.float32)
    cos, sin = pl.pallas_call(
        _rope_body,
        grid_spec=grid_spec,
        out_shape=[out_shape, out_shape],
    )(t_idxs.astype(jnp.int32), ttab_c, ttab_s, spat_c, spat_s)
    return (cos, sin)
